# baseline (speedup 1.0000x reference)
import jax
import jax.numpy as jnp
from jax import lax
from jax.experimental import pallas as pl
from jax.experimental.pallas import tpu as pltpu
from jax.experimental.pallas import tpu_sc as plsc

_NUM_CLASS = 5
_NBINS = _NUM_CLASS * _NUM_CLASS
_NPAIR = _NBINS * _NBINS
_LANES = 16
_NC = 2
_NS = 16
_NW = _NC * _NS
_ACC = _NPAIR * _LANES


def _label_kernel(pred_ref, tgt_ref, out_ref):
    p0 = pred_ref[0, 0]
    p1 = pred_ref[0, 1]
    p2 = pred_ref[0, 2]
    p3 = pred_ref[0, 3]
    p4 = pred_ref[0, 4]
    m01 = p1 > p0
    v01 = jnp.where(m01, p1, p0)
    b01 = jnp.where(m01, 1, 0)
    m23 = p3 > p2
    v23 = jnp.where(m23, p3, p2)
    b23 = jnp.where(m23, 3, 2)
    m03 = v23 > v01
    v03 = jnp.where(m03, v23, v01)
    b03 = jnp.where(m03, b23, b01)
    bi = jnp.where(p4 > v03, 4, b03)
    lab = tgt_ref[0] * _NUM_CLASS + bi
    half = lab.shape[0] // 2
    out_ref[0] = (lab[:half] * _NBINS + lab[half:]) * _LANES


def _pairs_tc(pred, target, block_h):
    n, _, h, w, d = pred.shape
    grid = (n, h // block_h)
    return pl.pallas_call(
        _label_kernel,
        grid=grid,
        in_specs=[
            pl.BlockSpec((1, _NUM_CLASS, block_h, w, d),
                         lambda i, j: (i, 0, j, 0, 0)),
            pl.BlockSpec((1, block_h, w, d), lambda i, j: (i, j, 0, 0)),
        ],
        out_specs=pl.BlockSpec((1, block_h // 2, w, d),
                               lambda i, j: (i, j, 0, 0)),
        out_shape=jax.ShapeDtypeStruct((n, h // 2, w, d), jnp.int32),
    )(pred, target)


def _sc_partial_counts(pairs_flat, n_pairs, chunk, unroll=16):
    per_worker = n_pairs // _NW
    n_chunks = per_worker // chunk
    vregs_per_chunk = chunk // _LANES

    mesh = plsc.VectorSubcoreMesh(
        core_axis_name="c", subcore_axis_name="s",
        num_cores=_NC, num_subcores=_NS)

    def body(lbl_hbm, out_hbm, lbuf_a, lbuf_b, acc, acc2, sem_a, sem_b):
        wid = lax.axis_index("s") * _NC + lax.axis_index("c")
        lane = lax.iota(jnp.int32, _LANES)
        ones = jnp.ones((_LANES,), jnp.float32)
        zeros = jnp.zeros((_LANES,), jnp.float32)
        banks = (acc, acc2)

        def zero_body(b, _):
            sl = pl.ds(b * _LANES, _LANES)
            acc[sl] = zeros
            acc2[sl] = zeros
            return 0

        lax.fori_loop(0, _NPAIR, zero_body, 0)

        vbase = wid * per_worker

        def issue(k, lbuf, sem):
            pltpu.async_copy(lbl_hbm.at[pl.ds(vbase + k * chunk, chunk)],
                             lbuf, sem)

        def drain(k, lbuf, sem):
            pltpu.make_async_copy(lbl_hbm.at[pl.ds(vbase + k * chunk, chunk)],
                                  lbuf, sem).wait()

        def compute(lbuf):
            def vreg_body(i, _):
                s0 = i * (_LANES * unroll)
                for u in range(unroll):
                    l16 = lbuf[pl.ds(s0 + u * _LANES, _LANES)]
                    plsc.addupdate_scatter(banks[u % 2], [l16 + lane], ones)
                return 0

            lax.fori_loop(0, vregs_per_chunk // unroll, vreg_body, 0)

        issue(0, lbuf_a, sem_a)
        n_half = n_chunks // 2

        def k2_body(k2, _):
            ka = 2 * k2
            issue(ka + 1, lbuf_b, sem_b)
            drain(ka, lbuf_a, sem_a)
            compute(lbuf_a)

            @pl.when(k2 < n_half - 1)
            def _prefetch():
                issue(ka + 2, lbuf_a, sem_a)

            drain(ka + 1, lbuf_b, sem_b)
            compute(lbuf_b)
            return 0

        lax.fori_loop(0, n_half, k2_body, 0)

        def merge_body(b, _):
            sl = pl.ds(b * _LANES, _LANES)
            acc[sl] = acc[sl] + acc2[sl]
            return 0

        lax.fori_loop(0, _NPAIR, merge_body, 0)
        pltpu.sync_copy(acc, out_hbm.at[pl.ds(wid * _ACC, _ACC)])

    return pl.kernel(
        body,
        out_type=jax.ShapeDtypeStruct((_NW * _ACC,), jnp.float32),
        mesh=mesh,
        compiler_params=pltpu.CompilerParams(needs_layout_passes=False),
        scratch_types=[
            pltpu.VMEM((chunk,), jnp.int32),
            pltpu.VMEM((chunk,), jnp.int32),
            pltpu.VMEM((_ACC,), jnp.float32),
            pltpu.VMEM((_ACC,), jnp.float32),
            pltpu.SemaphoreType.DMA,
            pltpu.SemaphoreType.DMA,
        ],
    )(pairs_flat)


def _finish_kernel(cnt_ref, out_ref):
    x = jnp.sum(cnt_ref[...], axis=0, keepdims=True)
    col = lax.broadcasted_iota(jnp.int32, x.shape, 1)
    pair = col // _LANES
    la = pair // _NBINS
    lb = pair - la * _NBINS
    ia = la // _NUM_CLASS
    ja = la - ia * _NUM_CLASS
    ib = lb // _NUM_CLASS
    jb = lb - ib * _NUM_CLASS
    lane = lax.broadcasted_iota(jnp.int32, (1, 128), 1)
    res = jnp.zeros((1, 128), jnp.float32)
    for cls in range(_NUM_CLASS):
        dmask = ((la == 6 * cls).astype(jnp.float32)
                 + (lb == 6 * cls).astype(jnp.float32))
        rmask = ((ia == cls).astype(jnp.float32)
                 + (ib == cls).astype(jnp.float32))
        cmask = ((ja == cls).astype(jnp.float32)
                 + (jb == cls).astype(jnp.float32))
        diag = jnp.sum(x * dmask)
        row = jnp.sum(x * rmask)
        colsum = jnp.sum(x * cmask)
        dice = 2.0 * diag / (row + colsum)
        res = res + jnp.where(lane == cls, dice, 0.0)
    out_ref[...] = res


def kernel(pred, target):
    n = pred.shape[0]
    vol = pred.shape[2] * pred.shape[3] * pred.shape[4]
    tgt = target.astype(jnp.int32)

    n_pairs = n * vol // 2
    pairs = _pairs_tc(pred, tgt, block_h=8)
    partials = _sc_partial_counts(pairs.reshape(n_pairs), n_pairs, chunk=16384)

    out = pl.pallas_call(
        _finish_kernel,
        out_shape=jax.ShapeDtypeStruct((1, 128), jnp.float32),
    )(partials.reshape(_NW, _ACC))
    return out[0, :_NUM_CLASS]


# speedup vs baseline: 1.0643x; 1.0643x over previous
import jax
import jax.numpy as jnp
from jax import lax
from jax.experimental import pallas as pl
from jax.experimental.pallas import tpu as pltpu
from jax.experimental.pallas import tpu_sc as plsc

_NUM_CLASS = 5
_NBINS = _NUM_CLASS * _NUM_CLASS
_NPAIR = _NBINS * _NBINS
_LANES = 16
_NC = 2
_NS = 16
_NW = _NC * _NS
_ACC = _NPAIR * _LANES


def _label_kernel(pred_ref, tgt_ref, out_ref):
    p0 = pred_ref[0, 0]
    p1 = pred_ref[0, 1]
    p2 = pred_ref[0, 2]
    p3 = pred_ref[0, 3]
    p4 = pred_ref[0, 4]
    m01 = p1 > p0
    v01 = jnp.where(m01, p1, p0)
    b01 = jnp.where(m01, 1, 0)
    m23 = p3 > p2
    v23 = jnp.where(m23, p3, p2)
    b23 = jnp.where(m23, 3, 2)
    m03 = v23 > v01
    v03 = jnp.where(m03, v23, v01)
    b03 = jnp.where(m03, b23, b01)
    bi = jnp.where(p4 > v03, 4, b03)
    lab = tgt_ref[0] * _NUM_CLASS + bi
    half = lab.shape[0] // 2
    out_ref[0] = (lab[:half] * _NBINS + lab[half:]) * _LANES


def _pairs_tc(pred, target, block_h):
    n, _, h, w, d = pred.shape
    grid = (n, h // block_h)
    return pl.pallas_call(
        _label_kernel,
        grid=grid,
        in_specs=[
            pl.BlockSpec((1, _NUM_CLASS, block_h, w, d),
                         lambda i, j: (i, 0, j, 0, 0)),
            pl.BlockSpec((1, block_h, w, d), lambda i, j: (i, j, 0, 0)),
        ],
        out_specs=pl.BlockSpec((1, block_h // 2, w, d),
                               lambda i, j: (i, j, 0, 0)),
        out_shape=jax.ShapeDtypeStruct((n, h // 2, w, d), jnp.int32),
    )(pred, target)


def _sc_partial_counts(pairs_flat, n_pairs, chunk, unroll=16):
    per_worker = n_pairs // _NW
    n_chunks = per_worker // chunk
    vregs_per_chunk = chunk // _LANES

    mesh = plsc.VectorSubcoreMesh(
        core_axis_name="c", subcore_axis_name="s",
        num_cores=_NC, num_subcores=_NS)

    def body(lbl_hbm, out_hbm, lbuf_a, lbuf_b, acc, acc2, sem_a, sem_b):
        wid = lax.axis_index("s") * _NC + lax.axis_index("c")
        lane = lax.iota(jnp.int32, _LANES)
        ones = jnp.ones((_LANES,), jnp.float32)
        zeros = jnp.zeros((_LANES,), jnp.float32)
        banks = (acc, acc2)

        def zero_body(b, _):
            sl = pl.ds(b * _LANES, _LANES)
            acc[sl] = zeros
            acc2[sl] = zeros
            return 0

        lax.fori_loop(0, _NPAIR, zero_body, 0)

        vbase = wid * per_worker

        def issue(k, lbuf, sem):
            pltpu.async_copy(lbl_hbm.at[pl.ds(vbase + k * chunk, chunk)],
                             lbuf, sem)

        def drain(k, lbuf, sem):
            pltpu.make_async_copy(lbl_hbm.at[pl.ds(vbase + k * chunk, chunk)],
                                  lbuf, sem).wait()

        def compute(lbuf):
            def vreg_body(i, _):
                s0 = i * (_LANES * unroll)
                for u in range(unroll):
                    l16 = lbuf[pl.ds(s0 + u * _LANES, _LANES)]
                    plsc.addupdate_scatter(banks[u % 2], [l16 + lane], ones)
                return 0

            lax.fori_loop(0, vregs_per_chunk // unroll, vreg_body, 0)

        issue(0, lbuf_a, sem_a)
        n_half = n_chunks // 2

        def k2_body(k2, _):
            ka = 2 * k2
            issue(ka + 1, lbuf_b, sem_b)
            drain(ka, lbuf_a, sem_a)
            compute(lbuf_a)

            @pl.when(k2 < n_half - 1)
            def _prefetch():
                issue(ka + 2, lbuf_a, sem_a)

            drain(ka + 1, lbuf_b, sem_b)
            compute(lbuf_b)
            return 0

        lax.fori_loop(0, n_half, k2_body, 0)

        def merge_body(b, _):
            sl = pl.ds(b * _LANES, _LANES)
            acc[sl] = acc[sl] + acc2[sl]
            return 0

        lax.fori_loop(0, _NPAIR, merge_body, 0)
        pltpu.sync_copy(acc, out_hbm.at[pl.ds(wid * _ACC, _ACC)])

    return pl.kernel(
        body,
        out_type=jax.ShapeDtypeStruct((_NW * _ACC,), jnp.float32),
        mesh=mesh,
        compiler_params=pltpu.CompilerParams(needs_layout_passes=False),
        scratch_types=[
            pltpu.VMEM((chunk,), jnp.int32),
            pltpu.VMEM((chunk,), jnp.int32),
            pltpu.VMEM((_ACC,), jnp.float32),
            pltpu.VMEM((_ACC,), jnp.float32),
            pltpu.SemaphoreType.DMA,
            pltpu.SemaphoreType.DMA,
        ],
    )(pairs_flat)


def _finish_kernel(cnt_ref, out_ref):
    x = jnp.sum(cnt_ref[...], axis=0, keepdims=True)
    col = lax.broadcasted_iota(jnp.int32, x.shape, 1)
    pair = col // _LANES
    la = pair // _NBINS
    lb = pair - la * _NBINS
    ia = la // _NUM_CLASS
    ja = la - ia * _NUM_CLASS
    ib = lb // _NUM_CLASS
    jb = lb - ib * _NUM_CLASS
    lane = lax.broadcasted_iota(jnp.int32, (1, 128), 1)
    res = jnp.zeros((1, 128), jnp.float32)
    for cls in range(_NUM_CLASS):
        dmask = ((la == 6 * cls).astype(jnp.float32)
                 + (lb == 6 * cls).astype(jnp.float32))
        rmask = ((ia == cls).astype(jnp.float32)
                 + (ib == cls).astype(jnp.float32))
        cmask = ((ja == cls).astype(jnp.float32)
                 + (jb == cls).astype(jnp.float32))
        diag = jnp.sum(x * dmask)
        row = jnp.sum(x * rmask)
        colsum = jnp.sum(x * cmask)
        dice = 2.0 * diag / (row + colsum)
        res = res + jnp.where(lane == cls, dice, 0.0)
    out_ref[...] = res


def kernel(pred, target):
    n = pred.shape[0]
    vol = pred.shape[2] * pred.shape[3] * pred.shape[4]
    tgt = target.astype(jnp.int32)

    n_pairs = n * vol // 2
    pairs = _pairs_tc(pred, tgt, block_h=16)
    partials = _sc_partial_counts(pairs.reshape(n_pairs), n_pairs, chunk=16384)

    out = pl.pallas_call(
        _finish_kernel,
        out_shape=jax.ShapeDtypeStruct((1, 128), jnp.float32),
    )(partials.reshape(_NW, _ACC))
    return out[0, :_NUM_CLASS]
